# baseline (speedup 1.0000x reference)
import functools

import jax
import jax.numpy as jnp
from jax import lax
from jax.experimental import pallas as pl
from jax.experimental.pallas import tpu as pltpu
from jax.experimental.pallas import tpu_sc as plsc


@functools.lru_cache(maxsize=None)
def _make_gather(V, D, B):
    info = plsc.get_sparse_core_info()
    nc, ns = info.num_cores, info.num_subcores
    nw = nc * ns
    assert B % nw == 0 and (B // nw) % 8 == 0
    b_per_w = B // nw
    mesh = plsc.VectorSubcoreMesh(core_axis_name="c", subcore_axis_name="s")

    @functools.partial(
        pl.kernel, mesh=mesh,
        out_type=jax.ShapeDtypeStruct((B, 2 * D), jnp.float32),
        compiler_params=pltpu.CompilerParams(use_tc_tiling_on_sc=True),
        scratch_types=[
            pltpu.VMEM((b_per_w,), jnp.int32),
            pltpu.VMEM((b_per_w, 2 * D), jnp.float32),
            pltpu.SemaphoreType.DMA,
        ],
    )
    def gather_kernel(table2_hbm, idx2_hbm, out_hbm, idx_v, rows_v, sem):
        wid = lax.axis_index("s") * nc + lax.axis_index("c")
        base = wid * b_per_w
        pltpu.sync_copy(idx2_hbm.at[pl.ds(base, b_per_w)], idx_v)
        pltpu.async_copy(table2_hbm.at[idx_v], rows_v, sem).wait()
        pltpu.sync_copy(rows_v, out_hbm.at[pl.ds(base, b_per_w)])

    return gather_kernel


def _dense_body(xp_ref, sub_ref, enc_ref, dec_ref, bias_ref,
                emb_ref, out_ref, loss_ref):
    d = emb_ref.shape[1]
    par = (sub_ref[:] & 1).astype(jnp.float32)
    x = xp_ref[:, d:] * par + xp_ref[:, :d] * (1.0 - par)
    emb_ref[:] = x
    enc = enc_ref[:]
    dec = dec_ref[:]
    h = lax.dot_general(x, enc, (((1,), (1,)), ((), ())),
                        preferred_element_type=jnp.float32)
    binary = (h >= 0).astype(jnp.float32)
    y = lax.dot_general(binary, dec, (((1,), (1,)), ((), ())),
                        preferred_element_type=jnp.float32)
    out_ref[:] = y + bias_ref[:]
    corr = lax.dot_general(dec, enc, (((1,), (0,)), ((), ())),
                           preferred_element_type=jnp.float32)
    n = corr.shape[0]
    eye = (lax.broadcasted_iota(jnp.int32, (n, n), 0)
           == lax.broadcasted_iota(jnp.int32, (n, n), 1)).astype(jnp.float32)
    diff = corr - eye
    loss_ref[0, 0] = jnp.sqrt(jnp.sum(diff * diff))


@functools.lru_cache(maxsize=None)
def _make_dense(B, D, H, interpret=False):
    return pl.pallas_call(
        _dense_body,
        out_shape=(jax.ShapeDtypeStruct((B, D), jnp.float32),
                   jax.ShapeDtypeStruct((B, D), jnp.float32),
                   jax.ShapeDtypeStruct((1, 1), jnp.float32)),
        in_specs=[pl.BlockSpec(memory_space=pltpu.VMEM)] * 5,
        out_specs=(pl.BlockSpec(memory_space=pltpu.VMEM),
                   pl.BlockSpec(memory_space=pltpu.VMEM),
                   pl.BlockSpec(memory_space=pltpu.SMEM)),
        interpret=interpret,
    )


def kernel(input, emb_table, enc_w, dec_w, dec_b):
    idx = input.astype(jnp.int32)
    (B,) = idx.shape
    V, D = emb_table.shape
    H = enc_w.shape[0]
    table2 = emb_table.reshape(V // 2, 2 * D)
    groups = _make_gather(V, D, B)(table2, idx >> 1)
    in_embed, out_embed, loss = _make_dense(B, D, H)(
        groups, idx.reshape(B, 1), enc_w, dec_w, dec_b.reshape(1, D))
    return in_embed, out_embed, loss.reshape(())


# speedup vs baseline: 1.5331x; 1.5331x over previous
import functools

import jax
import jax.numpy as jnp
from jax import lax
from jax.experimental import pallas as pl
from jax.experimental.pallas import tpu as pltpu
from jax.experimental.pallas import tpu_sc as plsc

_NBUF = 16


@functools.lru_cache(maxsize=None)
def _make_gather(V, D, B):
    info = plsc.get_sparse_core_info()
    nc, ns = info.num_cores, info.num_subcores
    nw = nc * ns
    assert B % nw == 0 and (B // nw) % _NBUF == 0
    bpw = B // nw
    mesh = plsc.VectorSubcoreMesh(core_axis_name="c", subcore_axis_name="s")

    @functools.partial(
        pl.kernel, mesh=mesh,
        out_type=jax.ShapeDtypeStruct((B, D), jnp.float32),
        compiler_params=pltpu.CompilerParams(
            use_tc_tiling_on_sc=True, needs_layout_passes=False),
        scratch_types=[
            pltpu.VMEM((bpw,), jnp.int32),
            pltpu.VMEM((_NBUF, 8, D), jnp.float32),
            pltpu.VMEM((bpw, D), jnp.float32),
            pltpu.SemaphoreType.DMA((_NBUF,)),
        ],
    )
    def gather_kernel(table_hbm, idx_hbm, out_hbm, idx_v, grp_v, sel_v, sems):
        wid = lax.axis_index("s") * nc + lax.axis_index("c")
        base = wid * bpw
        pltpu.sync_copy(idx_hbm.at[pl.ds(base, bpw)], idx_v)
        iota16 = lax.iota(jnp.int32, 16)

        def fire(slot, v):
            goff = pl.multiple_of((v >> 3) * 8, 8)
            pltpu.async_copy(table_hbm.at[pl.ds(goff, 8), :],
                             grp_v.at[slot], sems.at[slot])

        def drain(slot):
            pltpu.make_async_copy(table_hbm.at[pl.ds(0, 8), :],
                                  grp_v.at[slot], sems.at[slot]).wait()

        def extract(i, slot, v):
            r = iota16 * 0 + (v & 7)
            for g in range(D // 16):
                vals = plsc.load_gather(grp_v.at[slot], [r, iota16 + g * 16])
                plsc.store_scatter(sel_v, [iota16 * 0 + i, iota16 + g * 16],
                                   vals)

        def prime(gi, _):
            vs = idx_v[pl.ds(gi * 16, 16)]
            for j in range(16):
                fire(gi * 16 + j, vs[j])
            return ()

        lax.fori_loop(0, _NBUF // 16, prime, ())

        def step(gi, _):
            vs_nxt = idx_v[pl.ds(gi * 16 + _NBUF, 16)]
            vs_cur = idx_v[pl.ds(gi * 16, 16)]
            for j in range(16):
                i = gi * 16 + j
                slot = i % _NBUF
                drain(slot)
                extract(i, slot, vs_cur[j])
                fire(slot, vs_nxt[j])
            return ()

        lax.fori_loop(0, (bpw - _NBUF) // 16, step, ())

        def tail(gi, _):
            base_i = bpw - _NBUF + gi * 16
            vs_cur = idx_v[pl.ds(base_i, 16)]
            for j in range(16):
                i = base_i + j
                slot = i % _NBUF
                drain(slot)
                extract(i, slot, vs_cur[j])
            return ()

        lax.fori_loop(0, _NBUF // 16, tail, ())
        pltpu.sync_copy(sel_v, out_hbm.at[pl.ds(base, bpw)])

    return gather_kernel


def _dense_body(x_ref, enc_ref, dec_ref, bias_ref, out_ref, loss_ref):
    x = x_ref[:]
    enc = enc_ref[:]
    dec = dec_ref[:]
    h = lax.dot_general(x, enc, (((1,), (1,)), ((), ())),
                        preferred_element_type=jnp.float32)
    binary = (h >= 0).astype(jnp.float32)
    y = lax.dot_general(binary, dec, (((1,), (1,)), ((), ())),
                        preferred_element_type=jnp.float32)
    out_ref[:] = y + bias_ref[:]
    corr = lax.dot_general(dec, enc, (((1,), (0,)), ((), ())),
                           preferred_element_type=jnp.float32)
    n = corr.shape[0]
    eye = (lax.broadcasted_iota(jnp.int32, (n, n), 0)
           == lax.broadcasted_iota(jnp.int32, (n, n), 1)).astype(jnp.float32)
    diff = corr - eye
    loss_ref[0, 0] = jnp.sqrt(jnp.sum(diff * diff))


@functools.lru_cache(maxsize=None)
def _make_dense(B, D, H, interpret=False):
    return pl.pallas_call(
        _dense_body,
        out_shape=(jax.ShapeDtypeStruct((B, D), jnp.float32),
                   jax.ShapeDtypeStruct((1, 1), jnp.float32)),
        in_specs=[pl.BlockSpec(memory_space=pltpu.VMEM)] * 4,
        out_specs=(pl.BlockSpec(memory_space=pltpu.VMEM),
                   pl.BlockSpec(memory_space=pltpu.SMEM)),
        interpret=interpret,
    )


def kernel(input, emb_table, enc_w, dec_w, dec_b):
    idx = input.astype(jnp.int32)
    (B,) = idx.shape
    V, D = emb_table.shape
    H = enc_w.shape[0]
    in_embed = _make_gather(V, D, B)(emb_table, idx)
    out_embed, loss = _make_dense(B, D, H)(
        in_embed, enc_w, dec_w, dec_b.reshape(1, D))
    return in_embed, out_embed, loss.reshape(())
